# baseline (speedup 1.0000x reference)
import functools

import jax
import jax.numpy as jnp
from jax.experimental import pallas as pl
from jax.experimental.pallas import tpu as pltpu


def _se_kernel(x_ref, w1t_ref, b1_ref, w2t_ref, b2_ref, o_ref, *, inv_hw):
    x = x_ref[...].astype(jnp.float32)
    hw = o_ref.shape[2]

    avg = jnp.sum(x, axis=2) * inv_hw
    s = jnp.dot(avg, w1t_ref[...], preferred_element_type=jnp.float32)
    s = jnp.maximum(s + b1_ref[...], 0.0)
    t = jnp.dot(s, w2t_ref[...], preferred_element_type=jnp.float32)
    t = t + b2_ref[...]

    scale = jnp.clip(t + 3.0, 0.0, 6.0) * (1.0 / 6.0)

    o_ref[...] = (x[:, :, :hw] * scale[:, :, None]).astype(o_ref.dtype)


def kernel(x_nchw, w1t, b1, w2t, b2):
    B, C, H, W = x_nchw.shape
    HW = H * W
    HWp = ((HW + 127) // 128) * 128
    x_flat = x_nchw.reshape(B, C, HW)
    x_flat = jnp.pad(x_flat, ((0, 0), (0, 0), (0, HWp - HW)))

    Bt = max(1, min(B, 8))
    while B % Bt:
        Bt -= 1
    grid = (B // Bt,)

    full = lambda a: pl.BlockSpec(a.shape, lambda b: (0,) * a.ndim)

    out = pl.pallas_call(
        functools.partial(_se_kernel, inv_hw=1.0 / HW),
        out_shape=jax.ShapeDtypeStruct((B, C, HW), x_flat.dtype),
        grid=grid,
        in_specs=[
            pl.BlockSpec((Bt, C, HWp), lambda b: (b, 0, 0)),
            full(w1t), full(b1), full(w2t), full(b2),
        ],
        out_specs=pl.BlockSpec((Bt, C, HW), lambda b: (b, 0, 0)),
        compiler_params=pltpu.CompilerParams(
            dimension_semantics=("parallel",),
            vmem_limit_bytes=64 << 20),
    )(x_flat, w1t, b1, w2t, b2)

    return out.reshape(B, C, H, W)


# speedup vs baseline: 1.2445x; 1.2445x over previous
import functools

import jax
import jax.numpy as jnp
from jax.experimental import pallas as pl
from jax.experimental.pallas import tpu as pltpu


def _se_kernel(x_ref, w1t_ref, b1_ref, w2t_ref, b2_ref, o_ref, *, inv_hw):
    x = x_ref[...].astype(jnp.float32)
    hw = x_ref.shape[2]

    avg = jnp.sum(x, axis=2) * inv_hw
    s = jnp.dot(avg, w1t_ref[...], preferred_element_type=jnp.float32)
    s = jnp.maximum(s + b1_ref[...], 0.0)
    t = jnp.dot(s, w2t_ref[...], preferred_element_type=jnp.float32)
    t = t + b2_ref[...]

    scale = jnp.clip(t + 3.0, 0.0, 6.0) * (1.0 / 6.0)

    o_ref[:, :, :hw] = (x * scale[:, :, None]).astype(o_ref.dtype)


def kernel(x_nchw, w1t, b1, w2t, b2):
    B, C, H, W = x_nchw.shape
    HW = H * W
    HWp = ((HW + 127) // 128) * 128
    x_flat = x_nchw.reshape(B, C, HW)

    Bt = max(1, min(B, 8))
    while B % Bt:
        Bt -= 1
    grid = (B // Bt,)

    full = lambda a: pl.BlockSpec(a.shape, lambda b: (0,) * a.ndim)

    out = pl.pallas_call(
        functools.partial(_se_kernel, inv_hw=1.0 / HW),
        out_shape=jax.ShapeDtypeStruct((B, C, HWp), x_flat.dtype),
        grid=grid,
        in_specs=[
            pl.BlockSpec((Bt, C, HW), lambda b: (b, 0, 0)),
            full(w1t), full(b1), full(w2t), full(b2),
        ],
        out_specs=pl.BlockSpec((Bt, C, HWp), lambda b: (b, 0, 0)),
        compiler_params=pltpu.CompilerParams(
            dimension_semantics=("parallel",),
            vmem_limit_bytes=64 << 20),
    )(x_flat, w1t, b1, w2t, b2)

    if HWp != HW:
        out = out[:, :, :HW]
    return out.reshape(B, C, H, W)


# speedup vs baseline: 1.2567x; 1.0098x over previous
import functools

import jax
import jax.numpy as jnp
from jax.experimental import pallas as pl
from jax.experimental.pallas import tpu as pltpu


def _se_kernel(x_ref, w1t_ref, b1_ref, w2t_ref, b2_ref, o_ref, *, inv_hw):
    x = x_ref[...].astype(jnp.float32)
    hw = x_ref.shape[2]

    avg = jnp.sum(x, axis=2) * inv_hw
    s = jnp.dot(avg, w1t_ref[...], preferred_element_type=jnp.float32)
    s = jnp.maximum(s + b1_ref[...], 0.0)
    t = jnp.dot(s, w2t_ref[...], preferred_element_type=jnp.float32)
    t = t + b2_ref[...]

    scale = jnp.clip(t + 3.0, 0.0, 6.0) * (1.0 / 6.0)

    o_ref[:, :, :hw] = (x * scale[:, :, None]).astype(o_ref.dtype)


def kernel(x_nchw, w1t, b1, w2t, b2):
    B, C, H, W = x_nchw.shape
    HW = H * W
    HWp = ((HW + 127) // 128) * 128
    x_flat = x_nchw.reshape(B, C, HW)

    Bt = max(1, min(B, 16))
    while B % Bt:
        Bt -= 1
    grid = (B // Bt,)

    full = lambda a: pl.BlockSpec(a.shape, lambda b: (0,) * a.ndim)

    out = pl.pallas_call(
        functools.partial(_se_kernel, inv_hw=1.0 / HW),
        out_shape=jax.ShapeDtypeStruct((B, C, HWp), x_flat.dtype),
        grid=grid,
        in_specs=[
            pl.BlockSpec((Bt, C, HW), lambda b: (b, 0, 0)),
            full(w1t), full(b1), full(w2t), full(b2),
        ],
        out_specs=pl.BlockSpec((Bt, C, HWp), lambda b: (b, 0, 0)),
        compiler_params=pltpu.CompilerParams(
            dimension_semantics=("parallel",),
            vmem_limit_bytes=64 << 20),
    )(x_flat, w1t, b1, w2t, b2)

    if HWp != HW:
        out = out[:, :, :HW]
    return out.reshape(B, C, H, W)
